# Initial kernel scaffold; baseline (speedup 1.0000x reference)
import jax
import jax.numpy as jnp
from jax.experimental import pallas as pl


def kernel(features, edge_index, rel_type, weight, w_comp, bn_gamma, bn_beta):
    raise NotImplementedError("write your pallas kernel here")


# speedup vs baseline: 8.3254x; 8.3254x over previous
import functools

import jax
import jax.numpy as jnp
from jax import lax
from jax.experimental import pallas as pl
from jax.experimental.pallas import tpu as pltpu
from jax.experimental.pallas import tpu_sc as plsc

IN_FEAT = 128
OUT_FEAT = 128
NUM_RELS = 8
NUM_BASES = 4
N_NODES = 10000
N_EDGES = 320000

NC = 2
NS = 16
L = 16
NW = NC * NS
EPT = N_EDGES // NW
CH = 128
NCH = EPT // CH
REM = EPT - NCH * CH
HPAD = 10240
RPT = HPAD // NS
ZROWS = 64
RB = 1000


def _embed_matmul(S, w2):
    def body(s_ref, w_ref, o_ref):
        o_ref[...] = jnp.dot(s_ref[...], w_ref[...],
                             preferred_element_type=jnp.float32)

    return pl.pallas_call(
        body,
        out_shape=jax.ShapeDtypeStruct((NUM_RELS * IN_FEAT, OUT_FEAT),
                                       jnp.float32),
    )(S, w2)


def _sc_scatter(src, dst, rel, feat, embed, zeros_h):
    mesh = plsc.VectorSubcoreMesh(core_axis_name="c", subcore_axis_name="s")

    @functools.partial(
        pl.kernel,
        mesh=mesh,
        out_type=jax.ShapeDtypeStruct((NC, HPAD, OUT_FEAT), jnp.float32),
        scratch_types=[
            pltpu.VMEM((CH,), jnp.int32),
            pltpu.VMEM((CH,), jnp.int32),
            pltpu.VMEM((CH,), jnp.int32),
            pltpu.VMEM((CH,), jnp.int32),
            pltpu.VMEM((CH,), jnp.int32),
            pltpu.VMEM((CH, OUT_FEAT), jnp.float32),
            pltpu.VMEM((REM,), jnp.int32),
            pltpu.VMEM((REM,), jnp.int32),
            pltpu.VMEM((REM,), jnp.int32),
            pltpu.VMEM((REM,), jnp.int32),
            pltpu.VMEM((REM,), jnp.int32),
            pltpu.VMEM((REM, OUT_FEAT), jnp.float32),
            pltpu.VMEM((ZROWS, OUT_FEAT), jnp.float32),
            pltpu.VMEM_SHARED((HPAD, OUT_FEAT), jnp.float32),
            pltpu.SemaphoreType.DMA,
        ],
    )
    def k(src_h, dst_h, rel_h, feat_h, embed_h, zeros_hbm, out_h,
          src_v, dst_v, rel_v, fsrc_v, idx_v, rows_v,
          src_r, dst_r, rel_r, fsrc_r, idx_r, rows_r, zbuf, h_sh, sem):
        c = lax.axis_index("c")
        s = lax.axis_index("s")
        wid = s * NC + c

        pltpu.sync_copy(zeros_hbm, zbuf)

        row0 = s * RPT

        def zfill(q, carry):
            pltpu.sync_copy(zbuf, h_sh.at[pl.ds(row0 + q * ZROWS, ZROWS)])
            return carry

        lax.fori_loop(0, RPT // ZROWS, zfill, 0)
        plsc.subcore_barrier()

        base = wid * EPT

        def chunk(ci, carry):
            e0 = base + ci * CH
            pltpu.sync_copy(src_h.at[pl.ds(e0, CH)], src_v)
            pltpu.sync_copy(dst_h.at[pl.ds(e0, CH)], dst_v)
            pltpu.sync_copy(rel_h.at[pl.ds(e0, CH)], rel_v)
            pltpu.async_copy(feat_h.at[src_v], fsrc_v, sem).wait()
            for j in range(CH // L):
                sl = pl.ds(j * L, L)
                idx_v[sl] = rel_v[sl] * IN_FEAT + fsrc_v[sl]
            pltpu.async_copy(embed_h.at[idx_v], rows_v, sem).wait()
            pltpu.sync_copy(rows_v, h_sh.at[dst_v], add=True)
            return carry

        lax.fori_loop(0, NCH, chunk, 0)

        e0 = base + NCH * CH
        pltpu.sync_copy(src_h.at[pl.ds(e0, REM)], src_r)
        pltpu.sync_copy(dst_h.at[pl.ds(e0, REM)], dst_r)
        pltpu.sync_copy(rel_h.at[pl.ds(e0, REM)], rel_r)
        pltpu.async_copy(feat_h.at[src_r], fsrc_r, sem).wait()
        sl = pl.ds(0, L)
        idx_r[sl] = rel_r[sl] * IN_FEAT + fsrc_r[sl]
        pltpu.async_copy(embed_h.at[idx_r], rows_r, sem).wait()
        pltpu.sync_copy(rows_r, h_sh.at[dst_r], add=True)

        plsc.subcore_barrier()

        def outq(q, carry):
            r0 = row0 + q * ZROWS
            pltpu.sync_copy(h_sh.at[pl.ds(r0, ZROWS)], zbuf)
            pltpu.sync_copy(zbuf, out_h.at[c, pl.ds(r0, ZROWS)])
            return carry

        lax.fori_loop(0, RPT // ZROWS, outq, 0)

    return k(src, dst, rel, feat, embed, zeros_h)


def _reduce(partials):
    def body(p_ref, hsum_ref, stats_ref):
        i = pl.program_id(0)
        sblk = p_ref[0] + p_ref[1]
        hsum_ref[...] = sblk
        part = jnp.concatenate(
            [jnp.sum(sblk, axis=0, keepdims=True),
             jnp.sum(sblk * sblk, axis=0, keepdims=True),
             jnp.zeros((6, OUT_FEAT), jnp.float32)], axis=0)

        @pl.when(i == 0)
        def _():
            stats_ref[...] = jnp.zeros((8, OUT_FEAT), jnp.float32)

        stats_ref[...] += part

    return pl.pallas_call(
        body,
        grid=(N_NODES // RB,),
        in_specs=[pl.BlockSpec((NC, RB, OUT_FEAT), lambda i: (0, i, 0))],
        out_specs=[pl.BlockSpec((RB, OUT_FEAT), lambda i: (i, 0)),
                   pl.BlockSpec((8, OUT_FEAT), lambda i: (0, 0))],
        out_shape=[jax.ShapeDtypeStruct((N_NODES, OUT_FEAT), jnp.float32),
                   jax.ShapeDtypeStruct((8, OUT_FEAT), jnp.float32)],
    )(partials)


def _bn(hsum, stats, gamma, beta):
    def body(h_ref, st_ref, g_ref, b_ref, o_ref):
        mean = st_ref[0:1] * (1.0 / N_NODES)
        ex2 = st_ref[1:2] * (1.0 / N_NODES)
        var = ex2 - mean * mean
        inv = lax.rsqrt(var + 1e-5)
        o_ref[...] = (h_ref[...] - mean) * inv * g_ref[...] + b_ref[...]

    return pl.pallas_call(
        body,
        grid=(N_NODES // RB,),
        in_specs=[pl.BlockSpec((RB, OUT_FEAT), lambda i: (i, 0)),
                  pl.BlockSpec((8, OUT_FEAT), lambda i: (0, 0)),
                  pl.BlockSpec((1, OUT_FEAT), lambda i: (0, 0)),
                  pl.BlockSpec((1, OUT_FEAT), lambda i: (0, 0))],
        out_specs=pl.BlockSpec((RB, OUT_FEAT), lambda i: (i, 0)),
        out_shape=jax.ShapeDtypeStruct((N_NODES, OUT_FEAT), jnp.float32),
    )(hsum, stats, gamma, beta)


def kernel(features, edge_index, rel_type, weight, w_comp, bn_gamma, bn_beta):
    feat = features.astype(jnp.int32)
    src = edge_index[0].astype(jnp.int32)
    dst = edge_index[1].astype(jnp.int32)
    rel = rel_type.astype(jnp.int32)

    k = jnp.arange(NUM_RELS * IN_FEAT)
    r = k // IN_FEAT
    f = k % IN_FEAT
    i = 16 * r + f // 8
    j = f % 8
    S = jnp.zeros((NUM_RELS * IN_FEAT, NUM_BASES * IN_FEAT), jnp.float32)
    cols = i[:, None] * NUM_BASES + jnp.arange(NUM_BASES)[None, :]
    S = S.at[k[:, None], cols].set(w_comp[j])

    embed = _embed_matmul(
        S, weight.reshape(NUM_BASES * IN_FEAT, OUT_FEAT).astype(jnp.float32))

    zeros_h = jnp.zeros((ZROWS, OUT_FEAT), jnp.float32)
    partials = _sc_scatter(src, dst, rel, feat, embed, zeros_h)
    hsum, stats = _reduce(partials)
    return _bn(hsum, stats, bn_gamma.reshape(1, OUT_FEAT),
               bn_beta.reshape(1, OUT_FEAT))
